# baseline (speedup 1.0000x reference)
import numpy as np

import jax
import jax.numpy as jnp
from jax.experimental import pallas as pl
from jax.experimental.pallas import tpu as pltpu

H = W = 16
CIN = 3
KH = KW = 5
C1 = 10
C2 = 20
PH = 6
ROWS_X = H * W * CIN
K1 = KH * W
M1R = 2 * C1 * 8
P1R = C1 * 8
K2 = KH * P1R
M2R = 2 * C2


def _sel1():
    j = np.arange(KW)[:, None, None, None]
    e = np.arange(2)[None, :, None, None]
    pw = np.arange(8)[None, None, :, None]
    w = np.arange(W)[None, None, None, :]
    return (w == 2 * pw + e + j).astype(np.float32)


def _sel2():
    j = np.arange(KW)[:, None, None]
    e = np.arange(2)[None, :, None]
    pw = np.arange(8)[None, None, :]
    return (pw == e + j).astype(np.float32)


_S1 = _sel1()
_S2 = _sel2()


def _gather_idx1():
    e = np.arange(2)[:, None, None, None, None, None]
    c = np.arange(C1)[None, :, None, None, None, None]
    pw = np.arange(8)[None, None, :, None, None, None]
    i = np.arange(KH)[None, None, None, :, None, None]
    ci = np.arange(CIN)[None, None, None, None, :, None]
    w = np.arange(W)[None, None, None, None, None, :]
    j = w - 2 * pw - e
    ok = (j >= 0) & (j < KW)
    flat = ((i * KW + np.clip(j, 0, KW - 1)) * CIN + ci) * 16 + c
    flat = np.where(ok, flat, 127 * 16)
    return np.broadcast_to(flat, (2, C1, 8, KH, CIN, W)).reshape(M1R, KH * 48)


def _gather_idx2():
    e = np.arange(2)[:, None, None, None, None]
    c2 = np.arange(C2)[None, :, None, None, None]
    i = np.arange(KH)[None, None, :, None, None]
    c1 = np.arange(C1)[None, None, None, :, None]
    pw = np.arange(8)[None, None, None, None, :]
    j = pw - e
    ok = (j >= 0) & (j < KW)
    flat = (i * KW + np.clip(j, 0, KW - 1)) * (16 * 128) + c1 * 128 + c2
    flat = np.where(ok, flat, 15 * 128)
    return np.broadcast_to(flat, (2, C2, KH, C1, 8)).reshape(M2R, K2)


_IDX1 = _gather_idx1()
_IDX2 = _gather_idx2()


def _fused_body(x_ref, w1_ref, b1_ref, w2_ref, b2_ref, o_ref, xt_s, p1_s):
    t = jnp.tanh(x_ref[...]).astype(jnp.bfloat16)
    nb = t.shape[1]
    for ci in range(CIN):
        xt_s[:, 16 * ci:16 * (ci + 1), :] = (
            t[256 * ci:256 * (ci + 1), :].reshape(16, 16, nb))

    e1 = (jax.lax.broadcasted_iota(jnp.int32, (P1R, 16), 0) // 8
          == jax.lax.broadcasted_iota(jnp.int32, (P1R, 16), 1)
          ).astype(jnp.float32)
    e2 = (jax.lax.broadcasted_iota(jnp.int32, (C2, 128), 0)
          == jax.lax.broadcasted_iota(jnp.int32, (C2, 128), 1)
          ).astype(jnp.float32)
    b1c = jax.lax.dot_general(e1, b1_ref[...], (((1,), (1,)), ((), ())))
    b2c = jax.lax.dot_general(e2, b2_ref[...], (((1,), (1,)), ((), ())))

    w1m = w1_ref[...]
    for ph in range(PH):
        oh0 = 2 * ph
        a = jnp.dot(w1m, xt_s[oh0:oh0 + KH].reshape(KH * 48, nb),
                    preferred_element_type=jnp.float32)
        b = jnp.dot(w1m, xt_s[oh0 + 1:oh0 + 1 + KH].reshape(KH * 48, nb),
                    preferred_element_type=jnp.float32)
        m = jnp.maximum(a, b)
        p = jnp.maximum(m[0:P1R, :], m[P1R:2 * P1R, :])
        p1_s[pl.ds(P1R * ph, P1R), :] = jnp.maximum(
            p + b1c, 0.0).astype(jnp.bfloat16)

    w2m = w2_ref[...]
    o0 = jnp.dot(w2m, p1_s[pl.ds(0, K2), :],
                 preferred_element_type=jnp.float32)
    o1 = jnp.dot(w2m, p1_s[pl.ds(P1R, K2), :],
                 preferred_element_type=jnp.float32)
    m = jnp.maximum(o0, o1)
    m = jnp.maximum(m[0:C2, :], m[C2:2 * C2, :]) + b2c
    o_ref[...] = jnp.maximum(m, 0.0).T


def kernel(x, w1, b1, w2, b2):
    n = x.shape[0]
    for nb in (2048, 1024, 512, 256, 128, 32, 8):
        if n % nb == 0:
            break
    else:
        nb = n
    grid = n // nb

    x2d = jnp.transpose(x, (1, 2, 3, 0)).reshape(ROWS_X, n)

    w1d = jnp.take(w1.reshape(-1), jnp.asarray(_IDX1)).astype(jnp.bfloat16)
    w2d = jnp.take(w2.reshape(-1), jnp.asarray(_IDX2)).astype(jnp.bfloat16)

    out = pl.pallas_call(
        _fused_body,
        out_shape=jax.ShapeDtypeStruct((n, C2), jnp.float32),
        grid=(grid,),
        in_specs=[
            pl.BlockSpec((ROWS_X, nb), lambda s: (0, s)),
            pl.BlockSpec((M1R, KH * 48), lambda s: (0, 0)),
            pl.BlockSpec((1, 16), lambda s: (0, 0)),
            pl.BlockSpec((M2R, K2), lambda s: (0, 0)),
            pl.BlockSpec((1, 128), lambda s: (0, 0)),
        ],
        out_specs=pl.BlockSpec((nb, C2), lambda s: (s, 0)),
        scratch_shapes=[
            pltpu.VMEM((16, 48, nb), jnp.bfloat16),
            pltpu.VMEM((PH * P1R, nb), jnp.bfloat16),
        ],
        compiler_params=pltpu.CompilerParams(
            dimension_semantics=("parallel",)),
    )(x2d, w1d, b1, w2d, b2)

    return out.reshape(n, C2, 1, 1)


# speedup vs baseline: 10.8604x; 10.8604x over previous
import numpy as np

import jax
import jax.numpy as jnp
from jax.experimental import pallas as pl
from jax.experimental.pallas import tpu as pltpu

H = W = 16
CIN = 3
KH = KW = 5
C1 = 10
C2 = 20
PH = 6
ROWS_X = H * W * CIN
K1 = KH * W
M1R = 2 * C1 * 8
P1R = C1 * 8
K2 = KH * P1R
M2R = 2 * C2


def _sel1():
    j = np.arange(KW)[:, None, None, None]
    e = np.arange(2)[None, :, None, None]
    pw = np.arange(8)[None, None, :, None]
    w = np.arange(W)[None, None, None, :]
    return (w == 2 * pw + e + j).astype(np.float32)


def _sel2():
    j = np.arange(KW)[:, None, None]
    e = np.arange(2)[None, :, None]
    pw = np.arange(8)[None, None, :]
    return (pw == e + j).astype(np.float32)


_S1 = _sel1()
_S2 = _sel2()


def _fused_body(x_ref, w1_ref, b1_ref, w2_ref, b2_ref, o_ref, xt_s, p1_s):
    t = jnp.tanh(x_ref[...]).astype(jnp.bfloat16)
    nb = t.shape[1]
    for ci in range(CIN):
        xt_s[:, 16 * ci:16 * (ci + 1), :] = (
            t[256 * ci:256 * (ci + 1), :].reshape(16, 16, nb))

    e1 = (jax.lax.broadcasted_iota(jnp.int32, (P1R, 16), 0) // 8
          == jax.lax.broadcasted_iota(jnp.int32, (P1R, 16), 1)
          ).astype(jnp.float32)
    e2 = (jax.lax.broadcasted_iota(jnp.int32, (C2, 128), 0)
          == jax.lax.broadcasted_iota(jnp.int32, (C2, 128), 1)
          ).astype(jnp.float32)
    b1c = jax.lax.dot_general(e1, b1_ref[...], (((1,), (1,)), ((), ())))
    b2c = jax.lax.dot_general(e2, b2_ref[...], (((1,), (1,)), ((), ())))

    w1m = w1_ref[...]
    for ph in range(PH):
        oh0 = 2 * ph
        a = jnp.dot(w1m, xt_s[oh0:oh0 + KH].reshape(KH * 48, nb),
                    preferred_element_type=jnp.float32)
        b = jnp.dot(w1m, xt_s[oh0 + 1:oh0 + 1 + KH].reshape(KH * 48, nb),
                    preferred_element_type=jnp.float32)
        m = jnp.maximum(a, b)
        p = jnp.maximum(m[0:P1R, :], m[P1R:2 * P1R, :])
        p1_s[pl.ds(P1R * ph, P1R), :] = jnp.maximum(
            p + b1c, 0.0).astype(jnp.bfloat16)

    w2m = w2_ref[...]
    o0 = jnp.dot(w2m, p1_s[pl.ds(0, K2), :],
                 preferred_element_type=jnp.float32)
    o1 = jnp.dot(w2m, p1_s[pl.ds(P1R, K2), :],
                 preferred_element_type=jnp.float32)
    m = jnp.maximum(o0, o1)
    m = jnp.maximum(m[0:C2, :], m[C2:2 * C2, :]) + b2c
    o_ref[...] = jnp.maximum(m, 0.0).T


def kernel(x, w1, b1, w2, b2):
    n = x.shape[0]
    for nb in (2048, 1024, 512, 256, 128, 32, 8):
        if n % nb == 0:
            break
    else:
        nb = n
    grid = n // nb

    x2d = jnp.transpose(x, (1, 2, 3, 0)).reshape(ROWS_X, n)

    a1 = w1[:KH * KW * CIN, :C1].reshape(KH, KW, CIN * C1).astype(jnp.bfloat16)
    a2 = w2[:, :C1, :C2].reshape(KH, KW, C1 * C2).astype(jnp.bfloat16)
    acat = jnp.concatenate([a1, a2], axis=2)
    scat = np.concatenate(
        [_S1.reshape(KW, 256), _S2.reshape(KW, 16)], axis=1)
    o = jax.lax.dot_general(acat, jnp.asarray(scat, dtype=jnp.bfloat16),
                            (((1,), (0,)), ((), ())))
    w1d = o[:, :CIN * C1, :256].reshape(KH, CIN, C1, 2, 8, W)
    w1d = jnp.transpose(w1d, (3, 2, 4, 0, 1, 5)).reshape(M1R, KH * 48)
    w2d = o[:, CIN * C1:, 256:].reshape(KH, C1, C2, 2, 8)
    w2d = jnp.transpose(w2d, (3, 2, 0, 1, 4)).reshape(M2R, K2)

    out = pl.pallas_call(
        _fused_body,
        out_shape=jax.ShapeDtypeStruct((n, C2), jnp.float32),
        grid=(grid,),
        in_specs=[
            pl.BlockSpec((ROWS_X, nb), lambda s: (0, s)),
            pl.BlockSpec((M1R, KH * 48), lambda s: (0, 0)),
            pl.BlockSpec((1, 16), lambda s: (0, 0)),
            pl.BlockSpec((M2R, K2), lambda s: (0, 0)),
            pl.BlockSpec((1, 128), lambda s: (0, 0)),
        ],
        out_specs=pl.BlockSpec((nb, C2), lambda s: (s, 0)),
        scratch_shapes=[
            pltpu.VMEM((16, 48, nb), jnp.bfloat16),
            pltpu.VMEM((PH * P1R, nb), jnp.bfloat16),
        ],
        compiler_params=pltpu.CompilerParams(
            dimension_semantics=("parallel",)),
    )(x2d, w1d, b1, w2d, b2)

    return out.reshape(n, C2, 1, 1)


# speedup vs baseline: 10.8923x; 1.0029x over previous
import numpy as np

import jax
import jax.numpy as jnp
from jax.experimental import pallas as pl
from jax.experimental.pallas import tpu as pltpu

H = W = 16
CIN = 3
KH = KW = 5
C1 = 10
C2 = 20
PH = 6
ROWS_X = H * W * CIN
K1 = KH * W
M1R = 2 * C1 * 8
P1R = C1 * 8
K2 = KH * P1R
M2R = 2 * C2


def _sel1():
    j = np.arange(KW)[:, None, None, None]
    e = np.arange(2)[None, :, None, None]
    pw = np.arange(8)[None, None, :, None]
    w = np.arange(W)[None, None, None, :]
    return (w == 2 * pw + e + j).astype(np.float32)


def _sel2():
    j = np.arange(KW)[:, None, None]
    e = np.arange(2)[None, :, None]
    pw = np.arange(8)[None, None, :]
    return (pw == e + j).astype(np.float32)


_S1 = _sel1()
_S2 = _sel2()


def _fused_body(x_ref, w1_ref, b1_ref, w2_ref, b2_ref, o_ref, xt_s, p1_s):
    t = jnp.tanh(x_ref[...]).astype(jnp.bfloat16)
    nb = t.shape[1]
    for ci in range(CIN):
        xt_s[:, 16 * ci:16 * (ci + 1), :] = (
            t[256 * ci:256 * (ci + 1), :].reshape(16, 16, nb))

    e1 = (jax.lax.broadcasted_iota(jnp.int32, (P1R, 16), 0) // 8
          == jax.lax.broadcasted_iota(jnp.int32, (P1R, 16), 1)
          ).astype(jnp.float32)
    e2 = (jax.lax.broadcasted_iota(jnp.int32, (C2, 128), 0)
          == jax.lax.broadcasted_iota(jnp.int32, (C2, 128), 1)
          ).astype(jnp.float32)
    b1c = jax.lax.dot_general(e1, b1_ref[...], (((1,), (1,)), ((), ())))
    b2c = jax.lax.dot_general(e2, b2_ref[...], (((1,), (1,)), ((), ())))

    w1m = w1_ref[...]
    for ph in range(PH):
        oh0 = 2 * ph
        a = jnp.dot(w1m, xt_s[oh0:oh0 + KH].reshape(KH * 48, nb),
                    preferred_element_type=jnp.float32)
        b = jnp.dot(w1m, xt_s[oh0 + 1:oh0 + 1 + KH].reshape(KH * 48, nb),
                    preferred_element_type=jnp.float32)
        m = jnp.maximum(a, b)
        p = jnp.maximum(m[0:P1R, :], m[P1R:2 * P1R, :])
        p1_s[pl.ds(P1R * ph, P1R), :] = jnp.maximum(
            p + b1c, 0.0).astype(jnp.bfloat16)

    w2m = w2_ref[...]
    o0 = jnp.dot(w2m, p1_s[pl.ds(0, K2), :],
                 preferred_element_type=jnp.float32)
    o1 = jnp.dot(w2m, p1_s[pl.ds(P1R, K2), :],
                 preferred_element_type=jnp.float32)
    m = jnp.maximum(o0, o1)
    m = jnp.maximum(m[0:C2, :], m[C2:2 * C2, :]) + b2c
    o_ref[...] = jnp.maximum(m, 0.0).T


def kernel(x, w1, b1, w2, b2):
    n = x.shape[0]
    for nb in (2048, 1024, 512, 256, 128, 32, 8):
        if n % nb == 0:
            break
    else:
        nb = n
    grid = n // nb

    x2d = jnp.transpose(x, (1, 2, 3, 0)).reshape(ROWS_X, n)

    a1 = w1[:KH * KW * CIN, :C1].reshape(KH, KW, CIN * C1).astype(jnp.bfloat16)
    a2 = w2[:, :C1, :C2].reshape(KH, KW, C1 * C2).astype(jnp.bfloat16)
    acat = jnp.concatenate([a1, a2], axis=2)
    scat = np.concatenate(
        [_S1.reshape(KW, 256), _S2.reshape(KW, 16)], axis=1)
    o = jax.lax.dot_general(acat, jnp.asarray(scat, dtype=jnp.bfloat16),
                            (((1,), (0,)), ((), ())))
    w1d = o[:, :CIN * C1, :256].reshape(KH, CIN, C1, 2, 8, W)
    w1d = jnp.transpose(w1d, (3, 2, 4, 0, 1, 5)).reshape(M1R, KH * 48)
    w2d = o[:, CIN * C1:, 256:].reshape(KH, C1, C2, 2, 8)
    w2d = jnp.transpose(w2d, (3, 2, 0, 1, 4)).reshape(M2R, K2)

    out = pl.pallas_call(
        _fused_body,
        out_shape=jax.ShapeDtypeStruct((n, C2), jnp.float32),
        grid=(grid,),
        in_specs=[
            pl.BlockSpec((ROWS_X, nb), lambda s: (0, s)),
            pl.BlockSpec((M1R, KH * 48), lambda s: (0, 0)),
            pl.BlockSpec((1, 16), lambda s: (0, 0)),
            pl.BlockSpec((M2R, K2), lambda s: (0, 0)),
            pl.BlockSpec((1, 128), lambda s: (0, 0)),
        ],
        out_specs=pl.BlockSpec((nb, C2), lambda s: (s, 0)),
        scratch_shapes=[
            pltpu.VMEM((16, 48, nb), jnp.bfloat16),
            pltpu.VMEM((PH * P1R, nb), jnp.bfloat16),
        ],
        compiler_params=pltpu.CompilerParams(
            dimension_semantics=("parallel",),
            allow_input_fusion=[False, True, False, True, False]),
    )(x2d, w1d, b1, w2d, b2)

    return out.reshape(n, C2, 1, 1)
